# baseline (speedup 1.0000x reference)
import functools

import jax
import jax.numpy as jnp
from jax import lax
from jax.experimental import pallas as pl
from jax.experimental.pallas import tpu as pltpu
from jax.experimental.pallas import tpu_sc as plsc

NC = 2
NS = 16
LANES = 16
NW = NC * NS

VOCAB = 1000000
EMBED = 128
NUM_TAGS = 50
B = 4096
L = 200

B_PER_W = B // NW
HALF = L // 2
EV = EMBED // LANES


def _sc_pool(table, ids3):
  mesh = plsc.VectorSubcoreMesh(core_axis_name="c", subcore_axis_name="s")

  NSLOT = 6
  NH = 2 * B_PER_W
  NFULL = NH // NSLOT
  NTAIL = NH - NFULL * NSLOT

  @functools.partial(
      pl.kernel,
      mesh=mesh,
      out_type=jax.ShapeDtypeStruct((NW, B_PER_W, EMBED), jnp.float32),
      scratch_types=[
          pltpu.VMEM((NH, HALF), jnp.int32),
          pltpu.VMEM((HALF, EMBED), jnp.float32),
          pltpu.VMEM((HALF, EMBED), jnp.float32),
          pltpu.VMEM((HALF, EMBED), jnp.float32),
          pltpu.VMEM((HALF, EMBED), jnp.float32),
          pltpu.VMEM((HALF, EMBED), jnp.float32),
          pltpu.VMEM((HALF, EMBED), jnp.float32),
          pltpu.VMEM((B_PER_W, EMBED), jnp.float32),
          pltpu.SemaphoreType.DMA,
          pltpu.SemaphoreType.DMA,
          pltpu.SemaphoreType.DMA,
          pltpu.SemaphoreType.DMA,
          pltpu.SemaphoreType.DMA,
          pltpu.SemaphoreType.DMA,
      ],
  )
  def k(table_hbm, ids_hbm, out_hbm, ids_v, r0, r1, r2, r3, r4, r5, out_v,
        s0, s1, s2, s3, s4, s5):
    wid = lax.axis_index("s") * NC + lax.axis_index("c")
    rows = (r0, r1, r2, r3, r4, r5)
    sems = (s0, s1, s2, s3, s4, s5)

    def start(j, h):
      pltpu.async_copy(table_hbm.at[ids_v.at[h]], rows[j], sems[j])

    def wait(j):
      pltpu.make_async_copy(table_hbm.at[ids_v.at[0]], rows[j], sems[j]).wait()

    def partial_sum(j):
      r = rows[j]

      def body(i, acc):
        return tuple(
            acc[k_] + r[i, pl.ds(LANES * k_, LANES)] for k_ in range(EV))

      return lax.fori_loop(
          0, HALF, body,
          tuple(jnp.zeros((LANES,), jnp.float32) for _ in range(EV)),
          unroll=2)

    pltpu.sync_copy(ids_hbm.at[wid], ids_v)

    for j in range(NSLOT):
      start(j, j)

    def half_step(j, h, do_start):
      s = h // 2
      wait(j)
      acc = partial_sum(j)
      if j % 2 == 0:
        for k_ in range(EV):
          out_v[s, pl.ds(LANES * k_, LANES)] = acc[k_]
      else:
        for k_ in range(EV):
          sl = pl.ds(LANES * k_, LANES)
          out_v[s, sl] = (out_v[s, sl] + acc[k_]) * (1.0 / L)

      if do_start:
        @pl.when(h + NSLOT < NH)
        def _():
          start(j, h + NSLOT)

    def g_body(g, carry):
      for j in range(NSLOT):
        half_step(j, NSLOT * g + j, True)
      return carry

    lax.fori_loop(0, NFULL, g_body, 0)
    for j in range(NTAIL):
      half_step(j, NFULL * NSLOT + j, False)
    pltpu.sync_copy(out_v, out_hbm.at[wid])

  return k(table, ids3)


def _tc_head(x_ref, wt_ref, b_ref, o_ref):
  o_ref[...] = jax.nn.sigmoid(
      jnp.dot(x_ref[...], wt_ref[...], preferred_element_type=jnp.float32)
      + b_ref[...])


def kernel(input_ids, emb_table, W, b):
  ids3 = input_ids.reshape(NW, 2 * B_PER_W, HALF)
  pooled = _sc_pool(emb_table, ids3).reshape(B, EMBED)

  return pl.pallas_call(
      _tc_head,
      out_shape=jax.ShapeDtypeStruct((B, NUM_TAGS), jnp.float32),
  )(pooled, W.T, b.reshape(1, NUM_TAGS))


# speedup vs baseline: 1.0062x; 1.0062x over previous
import functools

import jax
import jax.numpy as jnp
from jax import lax
from jax.experimental import pallas as pl
from jax.experimental.pallas import tpu as pltpu
from jax.experimental.pallas import tpu_sc as plsc

NC = 2
NS = 16
LANES = 16
NW = NC * NS

VOCAB = 1000000
EMBED = 128
NUM_TAGS = 50
B = 4096
L = 200

B_PER_W = B // NW
HALF = L // 2
EV = EMBED // LANES


def _sc_pool(table, ids3):
  mesh = plsc.VectorSubcoreMesh(core_axis_name="c", subcore_axis_name="s")

  NSLOT = 6
  NH = 2 * B_PER_W
  NFULL = NH // NSLOT
  NTAIL = NH - NFULL * NSLOT

  @functools.partial(
      pl.kernel,
      mesh=mesh,
      out_type=jax.ShapeDtypeStruct((NW, B_PER_W, EMBED), jnp.float32),
      scratch_types=[
          pltpu.VMEM((NH, HALF), jnp.int32),
          pltpu.VMEM((HALF, EMBED), jnp.float32),
          pltpu.VMEM((HALF, EMBED), jnp.float32),
          pltpu.VMEM((HALF, EMBED), jnp.float32),
          pltpu.VMEM((HALF, EMBED), jnp.float32),
          pltpu.VMEM((HALF, EMBED), jnp.float32),
          pltpu.VMEM((HALF, EMBED), jnp.float32),
          pltpu.VMEM((B_PER_W, EMBED), jnp.float32),
          pltpu.SemaphoreType.DMA,
          pltpu.SemaphoreType.DMA,
          pltpu.SemaphoreType.DMA,
          pltpu.SemaphoreType.DMA,
          pltpu.SemaphoreType.DMA,
          pltpu.SemaphoreType.DMA,
      ],
  )
  def k(table_hbm, ids_hbm, out_hbm, ids_v, r0, r1, r2, r3, r4, r5, out_v,
        s0, s1, s2, s3, s4, s5):
    wid = lax.axis_index("s") * NC + lax.axis_index("c")
    rows = (r0, r1, r2, r3, r4, r5)
    sems = (s0, s1, s2, s3, s4, s5)

    def start(j, h):
      pltpu.async_copy(table_hbm.at[ids_v.at[h]], rows[j], sems[j])

    def wait(j):
      pltpu.make_async_copy(table_hbm.at[ids_v.at[0]], rows[j], sems[j]).wait()

    def partial_sum(j):
      r = rows[j]

      def body(i, acc):
        return tuple(
            acc[k_] + r[i, pl.ds(LANES * k_, LANES)] for k_ in range(EV))

      return lax.fori_loop(
          0, HALF, body,
          tuple(jnp.zeros((LANES,), jnp.float32) for _ in range(EV)),
          unroll=2)

    pltpu.sync_copy(ids_hbm.at[wid], ids_v)

    for j in range(NSLOT):
      start(j, j)

    def half_step(j, h, do_start):
      s = h // 2
      wait(j)
      acc = partial_sum(j)
      if j % 2 == 0:
        for k_ in range(EV):
          out_v[s, pl.ds(LANES * k_, LANES)] = acc[k_]
      else:
        for k_ in range(EV):
          sl = pl.ds(LANES * k_, LANES)
          out_v[s, sl] = (out_v[s, sl] + acc[k_]) * (1.0 / L)

      if do_start:
        @pl.when(h + NSLOT < NH)
        def _():
          start(j, h + NSLOT)

    def g_body(g, carry):
      for j in range(NSLOT):
        half_step(j, NSLOT * g + j, True)
      return carry

    lax.fori_loop(0, NFULL, g_body, 0)
    for j in range(NTAIL):
      half_step(j, NFULL * NSLOT + j, False)
    pltpu.sync_copy(out_v, out_hbm.at[wid])

  return k(table, ids3)


def _tc_head(x_ref, w_ref, b_ref, o_ref):
  logits = lax.dot_general(
      x_ref[...], w_ref[...], (((1,), (1,)), ((), ())),
      preferred_element_type=jnp.float32)
  o_ref[...] = jax.nn.sigmoid(logits + b_ref[...])


def kernel(input_ids, emb_table, W, b):
  ids3 = input_ids.reshape(NW, 2 * B_PER_W, HALF)
  pooled = _sc_pool(emb_table, ids3).reshape(B, EMBED)

  return pl.pallas_call(
      _tc_head,
      out_shape=jax.ShapeDtypeStruct((B, NUM_TAGS), jnp.float32),
  )(pooled, W, b.reshape(1, NUM_TAGS))
